# Initial kernel scaffold; baseline (speedup 1.0000x reference)
import jax
import jax.numpy as jnp
from jax.experimental import pallas as pl


def kernel(x, edge_index, W1, b1, W2, b2):
    raise NotImplementedError("write your pallas kernel here")


# speedup vs baseline: 3.3203x; 3.3203x over previous
import functools

import jax
import jax.numpy as jnp
from jax import lax
from jax.experimental import pallas as pl
from jax.experimental.pallas import tpu as pltpu
from jax.experimental.pallas import tpu_sc as plsc

N_NODES = 10000
N_EDGES = 320000
D_IN = 128
D_HID = 128
D_OUT = 64

NC = 2
NS = 16
NW = NC * NS
CHUNK = 128
N_CHUNKS = N_EDGES // (NW * CHUNK) + 2
E_PAD = NW * N_CHUNKS * CHUNK
N_PAD = 10240
RPT = N_PAD // NS
TRASH = N_NODES


def _mm1_body(x_ref, w_ref, o_ref):
    o_ref[...] = jnp.dot(x_ref[...], w_ref[...],
                         preferred_element_type=jnp.float32)


def _matmul1(x, w):
    blk = 1000
    return pl.pallas_call(
        _mm1_body,
        grid=(N_NODES // blk,),
        in_specs=[
            pl.BlockSpec((blk, D_IN), lambda i: (i, 0)),
            pl.BlockSpec((D_IN, D_HID), lambda i: (0, 0)),
        ],
        out_specs=pl.BlockSpec((blk, D_HID), lambda i: (i, 0)),
        out_shape=jax.ShapeDtypeStruct((N_NODES, D_HID), jnp.float32),
    )(x, w)


def _relu_body(p0_ref, p1_ref, b_ref, o_ref):
    o_ref[...] = jnp.maximum(p0_ref[0] + p1_ref[0] + b_ref[...], 0.0)


def _relu_add(partials, b1):
    blk = 1024
    return pl.pallas_call(
        _relu_body,
        grid=(N_PAD // blk,),
        in_specs=[
            pl.BlockSpec((1, blk, D_HID), lambda i: (0, i, 0)),
            pl.BlockSpec((1, blk, D_HID), lambda i: (1, i, 0)),
            pl.BlockSpec((1, D_HID), lambda i: (0, 0)),
        ],
        out_specs=pl.BlockSpec((blk, D_HID), lambda i: (i, 0)),
        out_shape=jax.ShapeDtypeStruct((N_PAD, D_HID), jnp.float32),
    )(partials, partials, b1.reshape(1, D_HID))


def _mm2_body(p0_ref, p1_ref, w_ref, b_ref, o_ref):
    agg = p0_ref[0] + p1_ref[0]
    o_ref[...] = jnp.dot(agg, w_ref[...],
                         preferred_element_type=jnp.float32) + b_ref[...]


def _final_matmul2(partials, w2, b2):
    blk = 1024
    return pl.pallas_call(
        _mm2_body,
        grid=(N_PAD // blk,),
        in_specs=[
            pl.BlockSpec((1, blk, D_HID), lambda i: (0, i, 0)),
            pl.BlockSpec((1, blk, D_HID), lambda i: (1, i, 0)),
            pl.BlockSpec((D_HID, D_OUT), lambda i: (0, 0)),
            pl.BlockSpec((1, D_OUT), lambda i: (0, 0)),
        ],
        out_specs=pl.BlockSpec((blk, D_OUT), lambda i: (i, 0)),
        out_shape=jax.ShapeDtypeStruct((N_PAD, D_OUT), jnp.float32),
    )(partials, partials, w2, b2.reshape(1, D_OUT))


def _make_spmm(d):
    mesh = plsc.VectorSubcoreMesh(core_axis_name="c", subcore_axis_name="s",
                                  num_cores=NC, num_subcores=NS)

    def body(table, srcs, dsts, zeros, out, src_v, dst_v, rows_v, acc, sem):
        c = lax.axis_index("c")
        s = lax.axis_index("s")
        wid = c * NS + s
        pltpu.sync_copy(zeros, acc.at[pl.ds(s * RPT, RPT)])
        pltpu.sync_copy(srcs.at[wid], src_v)
        pltpu.sync_copy(dsts.at[wid], dst_v)
        plsc.subcore_barrier()

        def step(j, carry):
            pltpu.async_copy(table.at[src_v.at[j]], rows_v, sem).wait()
            pltpu.sync_copy(rows_v, acc.at[dst_v.at[j]], add=True)
            return carry

        lax.fori_loop(0, N_CHUNKS, step, 0)
        plsc.subcore_barrier()
        pltpu.sync_copy(acc.at[pl.ds(s * RPT, RPT)],
                        out.at[c, pl.ds(s * RPT, RPT)])

    return pl.kernel(
        body,
        out_type=jax.ShapeDtypeStruct((NC, N_PAD, d), jnp.float32),
        mesh=mesh,
        scratch_types=[
            pltpu.VMEM((N_CHUNKS, CHUNK), jnp.int32),
            pltpu.VMEM((N_CHUNKS, CHUNK), jnp.int32),
            pltpu.VMEM((CHUNK, d), jnp.float32),
            pltpu.VMEM_SHARED((N_PAD, d), jnp.float32),
            pltpu.SemaphoreType.DMA,
        ],
    )


_spmm_hid = _make_spmm(D_HID)


def kernel(x, edge_index, W1, b1, W2, b2):
    src = edge_index[0].astype(jnp.int32)
    dst = edge_index[1].astype(jnp.int32)
    pad = E_PAD - N_EDGES
    srcs = jnp.concatenate([src, jnp.zeros((pad,), jnp.int32)])
    dsts = jnp.concatenate([dst, jnp.full((pad,), TRASH, jnp.int32)])
    srcs = srcs.reshape(NW, N_CHUNKS, CHUNK)
    dsts = dsts.reshape(NW, N_CHUNKS, CHUNK)
    zeros_hid = jnp.zeros((RPT, D_HID), jnp.float32)

    support = _matmul1(x, W1)
    partials1 = _spmm_hid(support, srcs, dsts, zeros_hid)
    h = _relu_add(partials1, b1)
    partials2 = _spmm_hid(h, srcs, dsts, zeros_hid)
    out = _final_matmul2(partials2, W2, b2)
    return out[:N_NODES]


# speedup vs baseline: 3.3793x; 1.0178x over previous
import functools

import jax
import jax.numpy as jnp
from jax import lax
from jax.experimental import pallas as pl
from jax.experimental.pallas import tpu as pltpu
from jax.experimental.pallas import tpu_sc as plsc

N_NODES = 10000
N_EDGES = 320000
D_IN = 128
D_HID = 128
D_OUT = 64

NC = 2
NS = 16
NW = NC * NS
CHUNK = 128
BLK = 8
NBLK = 10
N_CHUNKS = NBLK * BLK
E_PAD = NW * N_CHUNKS * CHUNK
N_PAD = 10240
RPT = N_PAD // NS
TRASH = N_NODES


def _mm1_body(x_ref, w_ref, o_ref):
    o_ref[...] = jnp.dot(x_ref[...], w_ref[...],
                         preferred_element_type=jnp.float32)


def _matmul1(x, w):
    blk = 1000
    return pl.pallas_call(
        _mm1_body,
        grid=(N_NODES // blk,),
        in_specs=[
            pl.BlockSpec((blk, D_IN), lambda i: (i, 0)),
            pl.BlockSpec((D_IN, D_HID), lambda i: (0, 0)),
        ],
        out_specs=pl.BlockSpec((blk, D_HID), lambda i: (i, 0)),
        out_shape=jax.ShapeDtypeStruct((N_NODES, D_HID), jnp.float32),
    )(x, w)


def _relu_body(p0_ref, p1_ref, b_ref, o_ref):
    o_ref[...] = jnp.maximum(p0_ref[0] + p1_ref[0] + b_ref[...], 0.0)


def _relu_add(partials, b1):
    blk = 1024
    return pl.pallas_call(
        _relu_body,
        grid=(N_PAD // blk,),
        in_specs=[
            pl.BlockSpec((1, blk, D_HID), lambda i: (0, i, 0)),
            pl.BlockSpec((1, blk, D_HID), lambda i: (1, i, 0)),
            pl.BlockSpec((1, D_HID), lambda i: (0, 0)),
        ],
        out_specs=pl.BlockSpec((blk, D_HID), lambda i: (i, 0)),
        out_shape=jax.ShapeDtypeStruct((N_PAD, D_HID), jnp.float32),
    )(partials, partials, b1.reshape(1, D_HID))


def _mm2_body(p0_ref, p1_ref, w_ref, b_ref, o_ref):
    agg = p0_ref[0] + p1_ref[0]
    o_ref[...] = jnp.dot(agg, w_ref[...],
                         preferred_element_type=jnp.float32) + b_ref[...]


def _final_matmul2(partials, w2, b2):
    blk = 1024
    return pl.pallas_call(
        _mm2_body,
        grid=(N_PAD // blk,),
        in_specs=[
            pl.BlockSpec((1, blk, D_HID), lambda i: (0, i, 0)),
            pl.BlockSpec((1, blk, D_HID), lambda i: (1, i, 0)),
            pl.BlockSpec((D_HID, D_OUT), lambda i: (0, 0)),
            pl.BlockSpec((1, D_OUT), lambda i: (0, 0)),
        ],
        out_specs=pl.BlockSpec((blk, D_OUT), lambda i: (i, 0)),
        out_shape=jax.ShapeDtypeStruct((N_PAD, D_OUT), jnp.float32),
    )(partials, partials, w2, b2.reshape(1, D_OUT))


def _make_spmm(d):
    mesh = plsc.VectorSubcoreMesh(core_axis_name="c", subcore_axis_name="s",
                                  num_cores=NC, num_subcores=NS)

    def body(table, srcs, dsts, out,
             sidx, didx, rows_v, acc, sem0, sem1):
        c = lax.axis_index("c")
        s = lax.axis_index("s")
        wid = c * NS + s
        zvec = jnp.zeros((16,), jnp.float32)

        def zstore(t, carry):
            rows_v[0, t // (d // 16), pl.ds((t % (d // 16)) * 16, 16)] = zvec
            return carry

        lax.fori_loop(0, CHUNK * d // 16, zstore, 0)

        def zdma(t, carry):
            pltpu.sync_copy(rows_v.at[0],
                            acc.at[pl.ds(s * RPT + t * CHUNK, CHUNK)])
            return carry

        lax.fori_loop(0, RPT // CHUNK, zdma, 0)
        plsc.subcore_barrier()

        sems = (sem0, sem1)

        def blk_body(b, carry):
            pltpu.sync_copy(srcs.at[wid, pl.ds(b * BLK, BLK)], sidx)
            pltpu.sync_copy(dsts.at[wid, pl.ds(b * BLK, BLK)], didx)
            pltpu.async_copy(table.at[sidx.at[0]], rows_v.at[0], sem0)
            for k in range(BLK):
                cur = k % 2
                if k + 1 < BLK:
                    pltpu.async_copy(table.at[sidx.at[k + 1]],
                                     rows_v.at[1 - cur], sems[1 - cur])
                pltpu.make_async_copy(table.at[sidx.at[k]],
                                      rows_v.at[cur], sems[cur]).wait()
                pltpu.sync_copy(rows_v.at[cur], acc.at[didx.at[k]], add=True)
            return carry

        lax.fori_loop(0, NBLK, blk_body, 0)
        plsc.subcore_barrier()
        pltpu.sync_copy(acc.at[pl.ds(s * RPT, RPT)],
                        out.at[c, pl.ds(s * RPT, RPT)])

    return pl.kernel(
        body,
        out_type=jax.ShapeDtypeStruct((NC, N_PAD, d), jnp.float32),
        mesh=mesh,
        scratch_types=[
            pltpu.VMEM((BLK, CHUNK), jnp.int32),
            pltpu.VMEM((BLK, CHUNK), jnp.int32),
            pltpu.VMEM((2, CHUNK, d), jnp.float32),
            pltpu.VMEM_SHARED((N_PAD, d), jnp.float32),
            pltpu.SemaphoreType.DMA,
            pltpu.SemaphoreType.DMA,
        ],
    )


_spmm_hid = _make_spmm(D_HID)


def kernel(x, edge_index, W1, b1, W2, b2):
    src = edge_index[0].astype(jnp.int32)
    dst = edge_index[1].astype(jnp.int32)
    pad = E_PAD - N_EDGES
    srcs = jnp.concatenate([src, jnp.zeros((pad,), jnp.int32)])
    dsts = jnp.concatenate([dst, jnp.full((pad,), TRASH, jnp.int32)])
    srcs = srcs.reshape(NW, N_CHUNKS, CHUNK)
    dsts = dsts.reshape(NW, N_CHUNKS, CHUNK)
    support = _matmul1(x, W1)
    partials1 = _spmm_hid(support, srcs, dsts)
    h = _relu_add(partials1, b1)
    partials2 = _spmm_hid(h, srcs, dsts)
    out = _final_matmul2(partials2, W2, b2)
    return out[:N_NODES]


# speedup vs baseline: 10.1730x; 3.0104x over previous
import functools

import jax
import jax.numpy as jnp
from jax import lax
from jax.experimental import pallas as pl
from jax.experimental.pallas import tpu as pltpu
from jax.experimental.pallas import tpu_sc as plsc

N_NODES = 10000
N_EDGES = 320000
D_IN = 128
D_HID = 128
D_OUT = 64

NC = 2
NS = 16
NW = NC * NS
CHUNK = 128
BLK = 8
NBLK = 10
N_CHUNKS = NBLK * BLK
E_PAD = NW * N_CHUNKS * CHUNK
N_PAD = 10240
RPT = N_PAD // NS
TRASH = N_NODES


def _mm1_body(x_ref, w_ref, o_ref):
    o_ref[...] = jnp.dot(x_ref[...], w_ref[...],
                         preferred_element_type=jnp.float32)


def _matmul1(x, w):
    blk = 1000
    return pl.pallas_call(
        _mm1_body,
        grid=(N_NODES // blk,),
        in_specs=[
            pl.BlockSpec((blk, D_IN), lambda i: (i, 0)),
            pl.BlockSpec((D_IN, D_HID), lambda i: (0, 0)),
        ],
        out_specs=pl.BlockSpec((blk, D_HID), lambda i: (i, 0)),
        out_shape=jax.ShapeDtypeStruct((N_NODES, D_HID), jnp.float32),
    )(x, w)


def _relu_body(p0_ref, p1_ref, b_ref, o_ref):
    o_ref[...] = jnp.maximum(p0_ref[0] + p1_ref[0] + b_ref[...], 0.0)


def _relu_add(partials, b1):
    blk = 1024
    return pl.pallas_call(
        _relu_body,
        grid=(N_PAD // blk,),
        in_specs=[
            pl.BlockSpec((1, blk, D_HID), lambda i: (0, i, 0)),
            pl.BlockSpec((1, blk, D_HID), lambda i: (1, i, 0)),
            pl.BlockSpec((1, D_HID), lambda i: (0, 0)),
        ],
        out_specs=pl.BlockSpec((blk, D_HID), lambda i: (i, 0)),
        out_shape=jax.ShapeDtypeStruct((N_PAD, D_HID), jnp.float32),
    )(partials, partials, b1.reshape(1, D_HID))


def _mm2_body(p0_ref, p1_ref, w_ref, b_ref, o_ref):
    agg = p0_ref[0] + p1_ref[0]
    o_ref[...] = jnp.dot(agg, w_ref[...],
                         preferred_element_type=jnp.float32) + b_ref[...]


def _final_matmul2(partials, w2, b2):
    blk = 1024
    return pl.pallas_call(
        _mm2_body,
        grid=(N_PAD // blk,),
        in_specs=[
            pl.BlockSpec((1, blk, D_HID), lambda i: (0, i, 0)),
            pl.BlockSpec((1, blk, D_HID), lambda i: (1, i, 0)),
            pl.BlockSpec((D_HID, D_OUT), lambda i: (0, 0)),
            pl.BlockSpec((1, D_OUT), lambda i: (0, 0)),
        ],
        out_specs=pl.BlockSpec((blk, D_OUT), lambda i: (i, 0)),
        out_shape=jax.ShapeDtypeStruct((N_PAD, D_OUT), jnp.float32),
    )(partials, partials, w2, b2.reshape(1, D_OUT))


def _make_spmm(d):
    mesh = plsc.VectorSubcoreMesh(core_axis_name="c", subcore_axis_name="s",
                                  num_cores=NC, num_subcores=NS)

    def body(table, srcs, dsts, out,
             sidx, didx, rows_v, acc, sem0, sem1):
        c = lax.axis_index("c")
        s = lax.axis_index("s")
        wid = c * NS + s
        zvec = jnp.zeros((16,), jnp.float32)

        def zstore(t, carry):
            rows_v[0, t // (d // 16), pl.ds((t % (d // 16)) * 16, 16)] = zvec
            return carry

        lax.fori_loop(0, CHUNK * d // 16, zstore, 0)

        def zdma(t, carry):
            pltpu.sync_copy(rows_v.at[0],
                            acc.at[pl.ds(s * RPT + t * CHUNK, CHUNK)])
            return carry

        lax.fori_loop(0, RPT // CHUNK, zdma, 0)
        plsc.subcore_barrier()

        sems = (sem0, sem1)

        def blk_body(b, carry):
            pltpu.sync_copy(srcs.at[wid, pl.ds(b * BLK, BLK)], sidx)
            pltpu.sync_copy(dsts.at[wid, pl.ds(b * BLK, BLK)], didx)
            pltpu.async_copy(table.at[sidx.at[0]], rows_v.at[0], sem0)
            for k in range(BLK):
                cur = k % 2
                if k + 1 < BLK:
                    pltpu.async_copy(table.at[sidx.at[k + 1]],
                                     rows_v.at[1 - cur], sems[1 - cur])
                pltpu.make_async_copy(table.at[sidx.at[k]],
                                      rows_v.at[cur], sems[cur]).wait()
                pltpu.sync_copy(rows_v.at[cur], acc.at[didx.at[k]], add=True)
            return carry

        lax.fori_loop(0, NBLK, blk_body, 0)
        plsc.subcore_barrier()
        pltpu.sync_copy(acc.at[pl.ds(s * RPT, RPT)],
                        out.at[c, pl.ds(s * RPT, RPT)])

    return pl.kernel(
        body,
        out_type=jax.ShapeDtypeStruct((NC, N_PAD, d), jnp.float32),
        mesh=mesh,
        scratch_types=[
            pltpu.VMEM((BLK, CHUNK), jnp.int32),
            pltpu.VMEM((BLK, CHUNK), jnp.int32),
            pltpu.VMEM((2, CHUNK, d), jnp.float32),
            pltpu.VMEM_SHARED((N_PAD, d), jnp.float32),
            pltpu.SemaphoreType.DMA,
            pltpu.SemaphoreType.DMA,
        ],
    )


_spmm_hid = _make_spmm(D_HID)


def kernel(x, edge_index, W1, b1, W2, b2):
    src = edge_index[0].astype(jnp.int32)
    dst = edge_index[1].astype(jnp.int32)
    pad = E_PAD - N_EDGES
    pad_ids = jnp.arange(pad, dtype=jnp.int32)
    srcs = jnp.concatenate([src, pad_ids % N_NODES])
    dsts = jnp.concatenate([dst, pad_ids % (N_PAD - N_NODES) + TRASH])
    srcs = srcs.reshape(NW, N_CHUNKS, CHUNK)
    dsts = dsts.reshape(NW, N_CHUNKS, CHUNK)
    support = _matmul1(x, W1)
    partials1 = _spmm_hid(support, srcs, dsts)
    h = _relu_add(partials1, b1)
    partials2 = _spmm_hid(h, srcs, dsts)
    out = _final_matmul2(partials2, W2, b2)
    return out[:N_NODES]


# speedup vs baseline: 11.5354x; 1.1339x over previous
import functools

import jax
import jax.numpy as jnp
from jax import lax
from jax.experimental import pallas as pl
from jax.experimental.pallas import tpu as pltpu
from jax.experimental.pallas import tpu_sc as plsc

N_NODES = 10000
N_EDGES = 320000
D_IN = 128
D_HID = 128
D_OUT = 64

NC = 2
NS = 16
NW = NC * NS
CHUNK = 128
BLK = 8
NBLK = 10
N_CHUNKS = NBLK * BLK
E_PAD = NW * N_CHUNKS * CHUNK
N_PAD = 10240
RPT = N_PAD // NS
TRASH = N_NODES


def _mm1_body(x_ref, w_ref, o_ref):
    o_ref[...] = jnp.dot(x_ref[...], w_ref[...],
                         preferred_element_type=jnp.float32)


def _matmul1(x, w):
    blk = 1000
    return pl.pallas_call(
        _mm1_body,
        grid=(N_NODES // blk,),
        in_specs=[
            pl.BlockSpec((blk, D_IN), lambda i: (i, 0)),
            pl.BlockSpec((D_IN, D_HID), lambda i: (0, 0)),
        ],
        out_specs=pl.BlockSpec((blk, D_HID), lambda i: (i, 0)),
        out_shape=jax.ShapeDtypeStruct((N_NODES, D_HID), jnp.float32),
    )(x, w)


def _relu_body(p0_ref, p1_ref, b_ref, o_ref):
    o_ref[...] = jnp.maximum(p0_ref[0] + p1_ref[0] + b_ref[...], 0.0)


def _relu_add(partials, b1):
    blk = 1024
    return pl.pallas_call(
        _relu_body,
        grid=(N_PAD // blk,),
        in_specs=[
            pl.BlockSpec((1, blk, D_HID), lambda i: (0, i, 0)),
            pl.BlockSpec((1, blk, D_HID), lambda i: (1, i, 0)),
            pl.BlockSpec((1, D_HID), lambda i: (0, 0)),
        ],
        out_specs=pl.BlockSpec((blk, D_HID), lambda i: (i, 0)),
        out_shape=jax.ShapeDtypeStruct((N_PAD, D_HID), jnp.float32),
    )(partials, partials, b1.reshape(1, D_HID))


def _mm2_body(p0_ref, p1_ref, w_ref, b_ref, o_ref):
    agg = p0_ref[0] + p1_ref[0]
    o_ref[...] = jnp.dot(agg, w_ref[...],
                         preferred_element_type=jnp.float32) + b_ref[...]


def _final_matmul2(partials, w2, b2):
    blk = 1024
    return pl.pallas_call(
        _mm2_body,
        grid=(N_PAD // blk,),
        in_specs=[
            pl.BlockSpec((1, blk, D_HID), lambda i: (0, i, 0)),
            pl.BlockSpec((1, blk, D_HID), lambda i: (1, i, 0)),
            pl.BlockSpec((D_HID, D_OUT), lambda i: (0, 0)),
            pl.BlockSpec((1, D_OUT), lambda i: (0, 0)),
        ],
        out_specs=pl.BlockSpec((blk, D_OUT), lambda i: (i, 0)),
        out_shape=jax.ShapeDtypeStruct((N_PAD, D_OUT), jnp.float32),
    )(partials, partials, w2, b2.reshape(1, D_OUT))


def _make_spmm(d):
    mesh = plsc.VectorSubcoreMesh(core_axis_name="c", subcore_axis_name="s",
                                  num_cores=NC, num_subcores=NS)

    def body(table, srcs, dsts, out,
             sidx, didx, rows_v, acc, sem0, sem1, semsi, semdi):
        c = lax.axis_index("c")
        s = lax.axis_index("s")
        wid = c * NS + s
        zvec = jnp.zeros((16,), jnp.float32)

        def zstore(t, carry):
            rows_v[0, t // (d // 16), pl.ds((t % (d // 16)) * 16, 16)] = zvec
            return carry

        lax.fori_loop(0, CHUNK * d // 16, zstore, 0)

        def zdma(t, carry):
            pltpu.sync_copy(rows_v.at[0],
                            acc.at[pl.ds(s * RPT + t * CHUNK, CHUNK)])
            return carry

        lax.fori_loop(0, RPT // CHUNK, zdma, 0)
        plsc.subcore_barrier()

        sems = (sem0, sem1)

        pltpu.sync_copy(srcs.at[wid, pl.ds(0, BLK)], sidx.at[0])
        pltpu.sync_copy(dsts.at[wid, pl.ds(0, BLK)], didx.at[0])
        pltpu.async_copy(table.at[sidx.at[0, 0]], rows_v.at[0], sem0)

        def blk_body(b, carry):
            sb = b % 2
            nb = 1 - sb

            @pl.when(b + 1 < NBLK)
            def _():
                pltpu.async_copy(srcs.at[wid, pl.ds((b + 1) * BLK, BLK)],
                                 sidx.at[nb], semsi)
                pltpu.async_copy(dsts.at[wid, pl.ds((b + 1) * BLK, BLK)],
                                 didx.at[nb], semdi)

            for k in range(BLK):
                cur = k % 2
                if k + 1 < BLK:
                    pltpu.async_copy(table.at[sidx.at[sb, k + 1]],
                                     rows_v.at[1 - cur], sems[1 - cur])
                else:
                    @pl.when(b + 1 < NBLK)
                    def _():
                        pltpu.make_async_copy(
                            srcs.at[wid, pl.ds((b + 1) * BLK, BLK)],
                            sidx.at[nb], semsi).wait()
                        pltpu.make_async_copy(
                            dsts.at[wid, pl.ds((b + 1) * BLK, BLK)],
                            didx.at[nb], semdi).wait()
                        pltpu.async_copy(table.at[sidx.at[nb, 0]],
                                         rows_v.at[1 - cur], sems[1 - cur])
                pltpu.make_async_copy(table.at[sidx.at[sb, k]],
                                      rows_v.at[cur], sems[cur]).wait()
                pltpu.sync_copy(rows_v.at[cur], acc.at[didx.at[sb, k]],
                                add=True)
            return carry

        lax.fori_loop(0, NBLK, blk_body, 0)
        plsc.subcore_barrier()
        pltpu.sync_copy(acc.at[pl.ds(s * RPT, RPT)],
                        out.at[c, pl.ds(s * RPT, RPT)])

    return pl.kernel(
        body,
        out_type=jax.ShapeDtypeStruct((NC, N_PAD, d), jnp.float32),
        mesh=mesh,
        scratch_types=[
            pltpu.VMEM((2, BLK, CHUNK), jnp.int32),
            pltpu.VMEM((2, BLK, CHUNK), jnp.int32),
            pltpu.VMEM((2, CHUNK, d), jnp.float32),
            pltpu.VMEM_SHARED((N_PAD, d), jnp.float32),
            pltpu.SemaphoreType.DMA,
            pltpu.SemaphoreType.DMA,
            pltpu.SemaphoreType.DMA,
            pltpu.SemaphoreType.DMA,
        ],
    )


_spmm_hid = _make_spmm(D_HID)


def kernel(x, edge_index, W1, b1, W2, b2):
    src = edge_index[0].astype(jnp.int32)
    dst = edge_index[1].astype(jnp.int32)
    pad = E_PAD - N_EDGES
    pad_ids = jnp.arange(pad, dtype=jnp.int32)
    srcs = jnp.concatenate([src, pad_ids % N_NODES])
    dsts = jnp.concatenate([dst, pad_ids % (N_PAD - N_NODES) + TRASH])
    srcs = srcs.reshape(NW, N_CHUNKS, CHUNK)
    dsts = dsts.reshape(NW, N_CHUNKS, CHUNK)
    support = _matmul1(x, W1)
    partials1 = _spmm_hid(support, srcs, dsts)
    h = _relu_add(partials1, b1)
    partials2 = _spmm_hid(h, srcs, dsts)
    out = _final_matmul2(partials2, W2, b2)
    return out[:N_NODES]


# speedup vs baseline: 12.3162x; 1.0677x over previous
import functools

import jax
import jax.numpy as jnp
from jax import lax
from jax.experimental import pallas as pl
from jax.experimental.pallas import tpu as pltpu
from jax.experimental.pallas import tpu_sc as plsc

N_NODES = 10000
N_EDGES = 320000
D_IN = 128
D_HID = 128
D_OUT = 64

NC = 2
NS = 16
NW = NC * NS
CHUNK = 112
BLK = 6
NBLK = 15
N_CHUNKS = NBLK * BLK
NBUF = 3
E_PAD = NW * N_CHUNKS * CHUNK
N_PAD = 10240
RPT = N_PAD // NS
TRASH = N_NODES


def _mm1_body(x_ref, w_ref, o_ref):
    o_ref[...] = jnp.dot(x_ref[...], w_ref[...],
                         preferred_element_type=jnp.float32)


def _matmul1(x, w):
    blk = 1000
    return pl.pallas_call(
        _mm1_body,
        grid=(N_NODES // blk,),
        in_specs=[
            pl.BlockSpec((blk, D_IN), lambda i: (i, 0)),
            pl.BlockSpec((D_IN, D_HID), lambda i: (0, 0)),
        ],
        out_specs=pl.BlockSpec((blk, D_HID), lambda i: (i, 0)),
        out_shape=jax.ShapeDtypeStruct((N_NODES, D_HID), jnp.float32),
    )(x, w)


def _relu_body(p0_ref, p1_ref, b_ref, o_ref):
    o_ref[...] = jnp.maximum(p0_ref[0] + p1_ref[0] + b_ref[...], 0.0)


def _relu_add(partials, b1):
    blk = 1024
    return pl.pallas_call(
        _relu_body,
        grid=(N_PAD // blk,),
        in_specs=[
            pl.BlockSpec((1, blk, D_HID), lambda i: (0, i, 0)),
            pl.BlockSpec((1, blk, D_HID), lambda i: (1, i, 0)),
            pl.BlockSpec((1, D_HID), lambda i: (0, 0)),
        ],
        out_specs=pl.BlockSpec((blk, D_HID), lambda i: (i, 0)),
        out_shape=jax.ShapeDtypeStruct((N_PAD, D_HID), jnp.float32),
    )(partials, partials, b1.reshape(1, D_HID))


def _mm2_body(p0_ref, p1_ref, w_ref, b_ref, o_ref):
    agg = p0_ref[0] + p1_ref[0]
    o_ref[...] = jnp.dot(agg, w_ref[...],
                         preferred_element_type=jnp.float32) + b_ref[...]


def _final_matmul2(partials, w2, b2):
    blk = 1024
    return pl.pallas_call(
        _mm2_body,
        grid=(N_PAD // blk,),
        in_specs=[
            pl.BlockSpec((1, blk, D_HID), lambda i: (0, i, 0)),
            pl.BlockSpec((1, blk, D_HID), lambda i: (1, i, 0)),
            pl.BlockSpec((D_HID, D_OUT), lambda i: (0, 0)),
            pl.BlockSpec((1, D_OUT), lambda i: (0, 0)),
        ],
        out_specs=pl.BlockSpec((blk, D_OUT), lambda i: (i, 0)),
        out_shape=jax.ShapeDtypeStruct((N_PAD, D_OUT), jnp.float32),
    )(partials, partials, w2, b2.reshape(1, D_OUT))


def _make_spmm(d):
    mesh = plsc.VectorSubcoreMesh(core_axis_name="c", subcore_axis_name="s",
                                  num_cores=NC, num_subcores=NS)

    def body(table, srcs, dsts, out, sidx, didx, rows_v, acc,
             gsem0, gsem1, gsem2, ssem0, ssem1, ssem2, semsi, semdi):
        c = lax.axis_index("c")
        s = lax.axis_index("s")
        wid = c * NS + s
        zvec = jnp.zeros((16,), jnp.float32)

        def zstore(t, carry):
            rows_v[0, t // (d // 16), pl.ds((t % (d // 16)) * 16, 16)] = zvec
            return carry

        lax.fori_loop(0, CHUNK * d // 16, zstore, 0)

        def zdma(t, carry):
            pltpu.sync_copy(rows_v.at[0, pl.ds(0, 64)],
                            acc.at[pl.ds(s * RPT + t * 64, 64)])
            return carry

        lax.fori_loop(0, RPT // 64, zdma, 0)
        plsc.subcore_barrier()

        gsems = (gsem0, gsem1, gsem2)
        ssems = (ssem0, ssem1, ssem2)

        pltpu.sync_copy(srcs.at[wid, 0], sidx.at[0])
        pltpu.sync_copy(dsts.at[wid, 0], didx.at[0])
        pltpu.async_copy(table.at[sidx.at[0, 0]], rows_v.at[0], gsem0)

        def blk_body(b, carry):
            sb = b % 2
            nb = 1 - sb

            for k in range(BLK):
                if k == 2:
                    @pl.when(b + 1 < NBLK)
                    def _():
                        pltpu.async_copy(srcs.at[wid, b + 1],
                                         sidx.at[nb], semsi)
                        pltpu.async_copy(dsts.at[wid, b + 1],
                                         didx.at[nb], semdi)
                cur = k % NBUF
                nxt = (k + 1) % NBUF
                if k >= 2:
                    pltpu.make_async_copy(
                        rows_v.at[nxt], acc.at[didx.at[sb, k - 2]],
                        ssems[nxt]).wait()
                else:
                    @pl.when(b > 0)
                    def _():
                        pltpu.make_async_copy(
                            rows_v.at[nxt], acc.at[didx.at[nb, BLK + k - 2]],
                            ssems[nxt]).wait()
                if k + 1 < BLK:
                    pltpu.async_copy(table.at[sidx.at[sb, k + 1]],
                                     rows_v.at[nxt], gsems[nxt])
                else:
                    @pl.when(b + 1 < NBLK)
                    def _():
                        pltpu.make_async_copy(srcs.at[wid, b + 1],
                                              sidx.at[nb], semsi).wait()
                        pltpu.make_async_copy(dsts.at[wid, b + 1],
                                              didx.at[nb], semdi).wait()
                        pltpu.async_copy(table.at[sidx.at[nb, 0]],
                                         rows_v.at[nxt], gsems[nxt])
                pltpu.make_async_copy(table.at[sidx.at[sb, k]],
                                      rows_v.at[cur], gsems[cur]).wait()
                pltpu.async_copy(rows_v.at[cur], acc.at[didx.at[sb, k]],
                                 ssems[cur], add=True)
            return carry

        lax.fori_loop(0, NBLK, blk_body, 0)
        lsb = (NBLK - 1) % 2
        pltpu.make_async_copy(rows_v.at[(BLK - 2) % NBUF],
                              acc.at[didx.at[lsb, BLK - 2]],
                              ssems[(BLK - 2) % NBUF]).wait()
        pltpu.make_async_copy(rows_v.at[(BLK - 1) % NBUF],
                              acc.at[didx.at[lsb, BLK - 1]],
                              ssems[(BLK - 1) % NBUF]).wait()
        plsc.subcore_barrier()
        pltpu.sync_copy(acc.at[pl.ds(s * RPT, RPT)],
                        out.at[c, pl.ds(s * RPT, RPT)])

    return pl.kernel(
        body,
        out_type=jax.ShapeDtypeStruct((NC, N_PAD, d), jnp.float32),
        mesh=mesh,
        scratch_types=[
            pltpu.VMEM((2, BLK, CHUNK), jnp.int32),
            pltpu.VMEM((2, BLK, CHUNK), jnp.int32),
            pltpu.VMEM((NBUF, CHUNK, d), jnp.float32),
            pltpu.VMEM_SHARED((N_PAD, d), jnp.float32),
            pltpu.SemaphoreType.DMA,
            pltpu.SemaphoreType.DMA,
            pltpu.SemaphoreType.DMA,
            pltpu.SemaphoreType.DMA,
            pltpu.SemaphoreType.DMA,
            pltpu.SemaphoreType.DMA,
            pltpu.SemaphoreType.DMA,
            pltpu.SemaphoreType.DMA,
        ],
    )


_spmm_hid = _make_spmm(D_HID)


def kernel(x, edge_index, W1, b1, W2, b2):
    src = edge_index[0].astype(jnp.int32)
    dst = edge_index[1].astype(jnp.int32)
    pad = E_PAD - N_EDGES
    pad_ids = jnp.arange(pad, dtype=jnp.int32)
    srcs = jnp.concatenate([src, pad_ids % N_NODES])
    dsts = jnp.concatenate([dst, pad_ids % (N_PAD - N_NODES) + TRASH])
    srcs = srcs.reshape(NW, NBLK, BLK, CHUNK)
    dsts = dsts.reshape(NW, NBLK, BLK, CHUNK)
    support = _matmul1(x, W1)
    partials1 = _spmm_hid(support, srcs, dsts)
    h = _relu_add(partials1, b1)
    partials2 = _spmm_hid(h, srcs, dsts)
    out = _final_matmul2(partials2, W2, b2)
    return out[:N_NODES]


# speedup vs baseline: 12.3565x; 1.0033x over previous
import functools

import jax
import jax.numpy as jnp
from jax import lax
from jax.experimental import pallas as pl
from jax.experimental.pallas import tpu as pltpu
from jax.experimental.pallas import tpu_sc as plsc

N_NODES = 10000
N_EDGES = 320000
D_IN = 128
D_HID = 128
D_OUT = 64

NC = 2
NS = 16
NW = NC * NS
CHUNK = 112
BLK = 6
NBLK = 15
N_CHUNKS = NBLK * BLK
NBUF = 3
E_PAD = NW * N_CHUNKS * CHUNK
N_PAD = 10240
RPT = N_PAD // NS
TRASH = N_NODES


def _mm1_body(x_ref, w_ref, o_ref):
    o_ref[...] = jnp.dot(x_ref[...], w_ref[...],
                         preferred_element_type=jnp.float32)


def _matmul1(x, w):
    blk = 1000
    return pl.pallas_call(
        _mm1_body,
        grid=(N_NODES // blk,),
        in_specs=[
            pl.BlockSpec((blk, D_IN), lambda i: (i, 0)),
            pl.BlockSpec((D_IN, D_HID), lambda i: (0, 0)),
        ],
        out_specs=pl.BlockSpec((blk, D_HID), lambda i: (i, 0)),
        out_shape=jax.ShapeDtypeStruct((N_NODES, D_HID), jnp.float32),
    )(x, w)


def _relu_body(p0_ref, p1_ref, b_ref, o_ref):
    o_ref[...] = jnp.maximum(p0_ref[0] + p1_ref[0] + b_ref[...], 0.0)


def _relu_add(partials, b1):
    blk = 1024
    return pl.pallas_call(
        _relu_body,
        grid=(N_PAD // blk,),
        in_specs=[
            pl.BlockSpec((1, blk, D_HID), lambda i: (0, i, 0)),
            pl.BlockSpec((1, blk, D_HID), lambda i: (1, i, 0)),
            pl.BlockSpec((1, D_HID), lambda i: (0, 0)),
        ],
        out_specs=pl.BlockSpec((blk, D_HID), lambda i: (i, 0)),
        out_shape=jax.ShapeDtypeStruct((N_PAD, D_HID), jnp.float32),
    )(partials, partials, b1.reshape(1, D_HID))


def _mm2_body(p0_ref, p1_ref, w_ref, b_ref, o_ref):
    agg = p0_ref[0] + p1_ref[0]
    o_ref[...] = jnp.dot(agg, w_ref[...],
                         preferred_element_type=jnp.float32) + b_ref[...]


def _final_matmul2(partials, w2, b2):
    blk = 1000
    return pl.pallas_call(
        _mm2_body,
        grid=(N_NODES // blk,),
        in_specs=[
            pl.BlockSpec((1, blk, D_HID), lambda i: (0, i, 0)),
            pl.BlockSpec((1, blk, D_HID), lambda i: (1, i, 0)),
            pl.BlockSpec((D_HID, D_OUT), lambda i: (0, 0)),
            pl.BlockSpec((1, D_OUT), lambda i: (0, 0)),
        ],
        out_specs=pl.BlockSpec((blk, D_OUT), lambda i: (i, 0)),
        out_shape=jax.ShapeDtypeStruct((N_NODES, D_OUT), jnp.float32),
    )(partials, partials, w2, b2.reshape(1, D_OUT))


def _make_spmm(d):
    mesh = plsc.VectorSubcoreMesh(core_axis_name="c", subcore_axis_name="s",
                                  num_cores=NC, num_subcores=NS)

    def body(table, srcs, dsts, out, sidx, didx, rows_v, acc,
             gsem0, gsem1, gsem2, ssem0, ssem1, ssem2, semsi, semdi):
        c = lax.axis_index("c")
        s = lax.axis_index("s")
        wid = c * NS + s
        zvec = jnp.zeros((16,), jnp.float32)

        def zstore(t, carry):
            rows_v[0, t // (d // 16), pl.ds((t % (d // 16)) * 16, 16)] = zvec
            return carry

        lax.fori_loop(0, CHUNK * d // 16, zstore, 0)

        def zfire(t, carry):
            pltpu.async_copy(rows_v.at[0, pl.ds(0, 64)],
                             acc.at[pl.ds(s * RPT + t * 64, 64)], gsem0)
            return carry

        lax.fori_loop(0, RPT // 64, zfire, 0)

        def zdrain(t, carry):
            pltpu.make_async_copy(rows_v.at[0, pl.ds(0, 64)],
                                  acc.at[pl.ds(s * RPT + t * 64, 64)],
                                  gsem0).wait()
            return carry

        lax.fori_loop(0, RPT // 64, zdrain, 0)
        plsc.subcore_barrier()

        gsems = (gsem0, gsem1, gsem2)
        ssems = (ssem0, ssem1, ssem2)

        pltpu.sync_copy(srcs.at[wid, 0], sidx.at[0])
        pltpu.sync_copy(dsts.at[wid, 0], didx.at[0])
        pltpu.async_copy(table.at[sidx.at[0, 0]], rows_v.at[0], gsem0)

        def blk_body(b, carry):
            sb = b % 2
            nb = 1 - sb

            for k in range(BLK):
                if k == 2:
                    @pl.when(b + 1 < NBLK)
                    def _():
                        pltpu.async_copy(srcs.at[wid, b + 1],
                                         sidx.at[nb], semsi)
                        pltpu.async_copy(dsts.at[wid, b + 1],
                                         didx.at[nb], semdi)
                cur = k % NBUF
                nxt = (k + 1) % NBUF
                if k >= 2:
                    pltpu.make_async_copy(
                        rows_v.at[nxt], acc.at[didx.at[sb, k - 2]],
                        ssems[nxt]).wait()
                else:
                    @pl.when(b > 0)
                    def _():
                        pltpu.make_async_copy(
                            rows_v.at[nxt], acc.at[didx.at[nb, BLK + k - 2]],
                            ssems[nxt]).wait()
                if k + 1 < BLK:
                    pltpu.async_copy(table.at[sidx.at[sb, k + 1]],
                                     rows_v.at[nxt], gsems[nxt])
                else:
                    @pl.when(b + 1 < NBLK)
                    def _():
                        pltpu.make_async_copy(srcs.at[wid, b + 1],
                                              sidx.at[nb], semsi).wait()
                        pltpu.make_async_copy(dsts.at[wid, b + 1],
                                              didx.at[nb], semdi).wait()
                        pltpu.async_copy(table.at[sidx.at[nb, 0]],
                                         rows_v.at[nxt], gsems[nxt])
                pltpu.make_async_copy(table.at[sidx.at[sb, k]],
                                      rows_v.at[cur], gsems[cur]).wait()
                pltpu.async_copy(rows_v.at[cur], acc.at[didx.at[sb, k]],
                                 ssems[cur], add=True)
            return carry

        lax.fori_loop(0, NBLK, blk_body, 0)
        lsb = (NBLK - 1) % 2
        pltpu.make_async_copy(rows_v.at[(BLK - 2) % NBUF],
                              acc.at[didx.at[lsb, BLK - 2]],
                              ssems[(BLK - 2) % NBUF]).wait()
        pltpu.make_async_copy(rows_v.at[(BLK - 1) % NBUF],
                              acc.at[didx.at[lsb, BLK - 1]],
                              ssems[(BLK - 1) % NBUF]).wait()
        plsc.subcore_barrier()
        pltpu.sync_copy(acc.at[pl.ds(s * RPT, RPT)],
                        out.at[c, pl.ds(s * RPT, RPT)])

    return pl.kernel(
        body,
        out_type=jax.ShapeDtypeStruct((NC, N_PAD, d), jnp.float32),
        mesh=mesh,
        scratch_types=[
            pltpu.VMEM((2, BLK, CHUNK), jnp.int32),
            pltpu.VMEM((2, BLK, CHUNK), jnp.int32),
            pltpu.VMEM((NBUF, CHUNK, d), jnp.float32),
            pltpu.VMEM_SHARED((N_PAD, d), jnp.float32),
            pltpu.SemaphoreType.DMA,
            pltpu.SemaphoreType.DMA,
            pltpu.SemaphoreType.DMA,
            pltpu.SemaphoreType.DMA,
            pltpu.SemaphoreType.DMA,
            pltpu.SemaphoreType.DMA,
            pltpu.SemaphoreType.DMA,
            pltpu.SemaphoreType.DMA,
        ],
    )


_spmm_hid = _make_spmm(D_HID)


def kernel(x, edge_index, W1, b1, W2, b2):
    src = edge_index[0].astype(jnp.int32)
    dst = edge_index[1].astype(jnp.int32)
    pad = E_PAD - N_EDGES
    pad_ids = jnp.arange(pad, dtype=jnp.int32)
    srcs = jnp.concatenate([src, pad_ids % N_NODES])
    dsts = jnp.concatenate([dst, pad_ids % (N_PAD - N_NODES) + TRASH])
    srcs = srcs.reshape(NW, NBLK, BLK, CHUNK)
    dsts = dsts.reshape(NW, NBLK, BLK, CHUNK)
    support = _matmul1(x, W1)
    partials1 = _spmm_hid(support, srcs, dsts)
    h = _relu_add(partials1, b1)
    partials2 = _spmm_hid(h, srcs, dsts)
    return _final_matmul2(partials2, W2, b2)


# speedup vs baseline: 12.7662x; 1.0332x over previous
import functools

import jax
import jax.numpy as jnp
from jax import lax
from jax.experimental import pallas as pl
from jax.experimental.pallas import tpu as pltpu
from jax.experimental.pallas import tpu_sc as plsc

N_NODES = 10000
N_EDGES = 320000
D_IN = 128
D_HID = 128
D_OUT = 64

NC = 2
NS = 16
NW = NC * NS
CHUNK = 112
BLK = 6
NBLK = 15
N_CHUNKS = NBLK * BLK
NBUF = 3
E_PAD = NW * N_CHUNKS * CHUNK
N_PAD = 10240
RPT = N_PAD // NS
TRASH = N_NODES


def _relu_mm_body(p0_ref, p1_ref, w_ref, b_ref, o_ref):
    agg = p0_ref[0] + p1_ref[0]
    o_ref[...] = jnp.maximum(
        jnp.dot(agg, w_ref[...], preferred_element_type=jnp.float32)
        + b_ref[...], 0.0)


def _relu_matmul1(partials, w1, b1):
    blk = 1024
    return pl.pallas_call(
        _relu_mm_body,
        grid=(N_PAD // blk,),
        in_specs=[
            pl.BlockSpec((1, blk, D_IN), lambda i: (0, i, 0)),
            pl.BlockSpec((1, blk, D_IN), lambda i: (1, i, 0)),
            pl.BlockSpec((D_IN, D_HID), lambda i: (0, 0)),
            pl.BlockSpec((1, D_HID), lambda i: (0, 0)),
        ],
        out_specs=pl.BlockSpec((blk, D_HID), lambda i: (i, 0)),
        out_shape=jax.ShapeDtypeStruct((N_PAD, D_HID), jnp.float32),
    )(partials, partials, w1, b1.reshape(1, D_HID))


def _mm2_body(p0_ref, p1_ref, w_ref, b_ref, o_ref):
    agg = p0_ref[0] + p1_ref[0]
    o_ref[...] = jnp.dot(agg, w_ref[...],
                         preferred_element_type=jnp.float32) + b_ref[...]


def _final_matmul2(partials, w2, b2):
    blk = 1000
    return pl.pallas_call(
        _mm2_body,
        grid=(N_NODES // blk,),
        in_specs=[
            pl.BlockSpec((1, blk, D_HID), lambda i: (0, i, 0)),
            pl.BlockSpec((1, blk, D_HID), lambda i: (1, i, 0)),
            pl.BlockSpec((D_HID, D_OUT), lambda i: (0, 0)),
            pl.BlockSpec((1, D_OUT), lambda i: (0, 0)),
        ],
        out_specs=pl.BlockSpec((blk, D_OUT), lambda i: (i, 0)),
        out_shape=jax.ShapeDtypeStruct((N_NODES, D_OUT), jnp.float32),
    )(partials, partials, w2, b2.reshape(1, D_OUT))


def _make_spmm(d):
    mesh = plsc.VectorSubcoreMesh(core_axis_name="c", subcore_axis_name="s",
                                  num_cores=NC, num_subcores=NS)

    def body(table, srcs, dsts, out, sidx, didx, rows_v, acc,
             gsem0, gsem1, gsem2, ssem0, ssem1, ssem2, semsi, semdi):
        c = lax.axis_index("c")
        s = lax.axis_index("s")
        wid = c * NS + s
        zvec = jnp.zeros((16,), jnp.float32)

        def zstore(t, carry):
            rows_v[0, t // (d // 16), pl.ds((t % (d // 16)) * 16, 16)] = zvec
            return carry

        lax.fori_loop(0, CHUNK * d // 16, zstore, 0)

        def zfire(t, carry):
            pltpu.async_copy(rows_v.at[0, pl.ds(0, 64)],
                             acc.at[pl.ds(s * RPT + t * 64, 64)], gsem0)
            return carry

        lax.fori_loop(0, RPT // 64, zfire, 0)

        def zdrain(t, carry):
            pltpu.make_async_copy(rows_v.at[0, pl.ds(0, 64)],
                                  acc.at[pl.ds(s * RPT + t * 64, 64)],
                                  gsem0).wait()
            return carry

        lax.fori_loop(0, RPT // 64, zdrain, 0)
        plsc.subcore_barrier()

        gsems = (gsem0, gsem1, gsem2)
        ssems = (ssem0, ssem1, ssem2)

        pltpu.sync_copy(srcs.at[wid, 0], sidx.at[0])
        pltpu.sync_copy(dsts.at[wid, 0], didx.at[0])
        pltpu.async_copy(table.at[sidx.at[0, 0]], rows_v.at[0], gsem0)

        def blk_body(b, carry):
            sb = b % 2
            nb = 1 - sb

            for k in range(BLK):
                if k == 2:
                    @pl.when(b + 1 < NBLK)
                    def _():
                        pltpu.async_copy(srcs.at[wid, b + 1],
                                         sidx.at[nb], semsi)
                        pltpu.async_copy(dsts.at[wid, b + 1],
                                         didx.at[nb], semdi)
                cur = k % NBUF
                nxt = (k + 1) % NBUF
                if k >= 2:
                    pltpu.make_async_copy(
                        rows_v.at[nxt], acc.at[didx.at[sb, k - 2]],
                        ssems[nxt]).wait()
                else:
                    @pl.when(b > 0)
                    def _():
                        pltpu.make_async_copy(
                            rows_v.at[nxt], acc.at[didx.at[nb, BLK + k - 2]],
                            ssems[nxt]).wait()
                if k + 1 < BLK:
                    pltpu.async_copy(table.at[sidx.at[sb, k + 1]],
                                     rows_v.at[nxt], gsems[nxt])
                else:
                    @pl.when(b + 1 < NBLK)
                    def _():
                        pltpu.make_async_copy(srcs.at[wid, b + 1],
                                              sidx.at[nb], semsi).wait()
                        pltpu.make_async_copy(dsts.at[wid, b + 1],
                                              didx.at[nb], semdi).wait()
                        pltpu.async_copy(table.at[sidx.at[nb, 0]],
                                         rows_v.at[nxt], gsems[nxt])
                pltpu.make_async_copy(table.at[sidx.at[sb, k]],
                                      rows_v.at[cur], gsems[cur]).wait()
                pltpu.async_copy(rows_v.at[cur], acc.at[didx.at[sb, k]],
                                 ssems[cur], add=True)
            return carry

        lax.fori_loop(0, NBLK, blk_body, 0)
        lsb = (NBLK - 1) % 2
        pltpu.make_async_copy(rows_v.at[(BLK - 2) % NBUF],
                              acc.at[didx.at[lsb, BLK - 2]],
                              ssems[(BLK - 2) % NBUF]).wait()
        pltpu.make_async_copy(rows_v.at[(BLK - 1) % NBUF],
                              acc.at[didx.at[lsb, BLK - 1]],
                              ssems[(BLK - 1) % NBUF]).wait()
        plsc.subcore_barrier()
        pltpu.sync_copy(acc.at[pl.ds(s * RPT, RPT)],
                        out.at[c, pl.ds(s * RPT, RPT)])

    return pl.kernel(
        body,
        out_type=jax.ShapeDtypeStruct((NC, N_PAD, d), jnp.float32),
        mesh=mesh,
        scratch_types=[
            pltpu.VMEM((2, BLK, CHUNK), jnp.int32),
            pltpu.VMEM((2, BLK, CHUNK), jnp.int32),
            pltpu.VMEM((NBUF, CHUNK, d), jnp.float32),
            pltpu.VMEM_SHARED((N_PAD, d), jnp.float32),
            pltpu.SemaphoreType.DMA,
            pltpu.SemaphoreType.DMA,
            pltpu.SemaphoreType.DMA,
            pltpu.SemaphoreType.DMA,
            pltpu.SemaphoreType.DMA,
            pltpu.SemaphoreType.DMA,
            pltpu.SemaphoreType.DMA,
            pltpu.SemaphoreType.DMA,
        ],
    )


_spmm_hid = _make_spmm(D_HID)


def kernel(x, edge_index, W1, b1, W2, b2):
    src = edge_index[0].astype(jnp.int32)
    dst = edge_index[1].astype(jnp.int32)
    pad = E_PAD - N_EDGES
    pad_ids = jnp.arange(pad, dtype=jnp.int32)
    srcs = jnp.concatenate([src, pad_ids % N_NODES])
    dsts = jnp.concatenate([dst, pad_ids % (N_PAD - N_NODES) + TRASH])
    srcs = srcs.reshape(NW, NBLK, BLK, CHUNK)
    dsts = dsts.reshape(NW, NBLK, BLK, CHUNK)
    partials1 = _spmm_hid(x, srcs, dsts)
    h = _relu_matmul1(partials1, W1, b1)
    partials2 = _spmm_hid(h, srcs, dsts)
    return _final_matmul2(partials2, W2, b2)
